# baseline (device time: 52279 ns/iter reference)
import jax
import jax.numpy as jnp
from jax import lax
from jax.experimental import pallas as pl
from jax.experimental.pallas import tpu as pltpu

N_DEV = 4
B = 2
SQ = 256
SKV_SH = 256
SKV = N_DEV * SKV_SH
H_LOC = 4
DH = 64
D_MODEL = 512

G = 32
BAND = 384

SEND_ORDER = (2, 1, 3)
RECV_ORDER = (1, 3, 2)


def kernel(x, Wq, K_ext, V_ext, Wo):
    x2d = x.reshape(B * SQ, D_MODEL)
    K_c = K_ext.transpose(0, 2, 1, 3).astype(jnp.bfloat16)
    V_c = V_ext.transpose(0, 2, 1, 3).astype(jnp.bfloat16)

    def body(x_ref, wq_ref, k_ref, v_ref, wo_ref, out_ref,
             kbuf, vbuf, pbuf, accbuf,
             send_k, recv_k, send_v, recv_v, send_p, recv_p):
        my = lax.axis_index("i")

        bsem = pltpu.get_barrier_semaphore()
        for k in range(1, N_DEV):
            pl.semaphore_signal(
                bsem, inc=1,
                device_id=((my + k) % N_DEV,),
                device_id_type=pl.DeviceIdType.MESH,
            )
        pl.semaphore_wait(bsem, N_DEV - 1)

        rdmas = {}
        for k in SEND_ORDER:
            dest = (my + k) % N_DEV
            rk = pltpu.make_async_remote_copy(
                src_ref=k_ref.at[:, pl.ds(dest * H_LOC, H_LOC), :, :],
                dst_ref=kbuf.at[k - 1],
                send_sem=send_k.at[k - 1],
                recv_sem=recv_k.at[k - 1],
                device_id=(dest,),
                device_id_type=pl.DeviceIdType.MESH,
            )
            rk.start()
            rv = pltpu.make_async_remote_copy(
                src_ref=v_ref.at[:, pl.ds(dest * H_LOC, H_LOC), :, :],
                dst_ref=vbuf.at[k - 1],
                send_sem=send_v.at[k - 1],
                recv_sem=recv_v.at[k - 1],
                device_id=(dest,),
                device_id_type=pl.DeviceIdType.MESH,
            )
            rv.start()
            rdmas[k] = (rk, rv)

        qb = lax.dot_general(
            x_ref[...].astype(jnp.bfloat16), wq_ref[...].astype(jnp.bfloat16),
            (((1,), (0,)), ((), ())),
            preferred_element_type=jnp.float32,
        ).astype(jnp.bfloat16)

        def local_k(b, h):
            return k_ref[b, my * H_LOC + h]

        def local_v(b, h):
            return v_ref[b, my * H_LOC + h]

        accg = [[None] * H_LOC for _ in range(B)]
        deng = [[None] * H_LOC for _ in range(B)]

        def glob_rows(get_k, get_v, first):
            for b in range(B):
                for h in range(H_LOC):
                    qg = qb[b * SQ:b * SQ + G, h * DH:(h + 1) * DH]
                    s = lax.dot_general(
                        qg, get_k(b, h), (((1,), (1,)), ((), ())),
                        preferred_element_type=jnp.float32,
                    ) * 0.125
                    p = jnp.exp(s)
                    pv = jnp.dot(p.astype(jnp.bfloat16), get_v(b, h),
                                 preferred_element_type=jnp.float32)
                    rs = jnp.sum(p, axis=1, keepdims=True)
                    if first:
                        accg[b][h] = pv
                        deng[b][h] = rs
                    else:
                        accg[b][h] = accg[b][h] + pv
                        deng[b][h] = deng[b][h] + rs

        glob_rows(local_k, local_v, first=True)
        for k in RECV_ORDER:
            rk, rv = rdmas[k]
            rk.wait_recv()
            rv.wait_recv()
            glob_rows(lambda b, h, _k=k: kbuf[_k - 1, b, h],
                      lambda b, h, _k=k: vbuf[_k - 1, b, h],
                      first=False)

        s0 = jnp.maximum((my - 0) % N_DEV - 1, 0)
        s1 = jnp.maximum((my - 1) % N_DEV - 1, 0)

        qi3 = G + lax.broadcasted_iota(jnp.int32, (SQ - G, BAND), 0)
        ki3 = lax.broadcasted_iota(jnp.int32, (SQ - G, BAND), 1)
        band_mask = (jnp.abs(qi3 - ki3) <= 128) | (ki3 < G)

        ctx_rows = []
        for b in range(B):
            ctx_cols = []
            for h in range(H_LOC):
                k0 = jnp.where(my == 0, local_k(b, h), kbuf[s0, b, h])
                k1 = jnp.where(my == 1, local_k(b, h), kbuf[s1, b, h])
                v0 = jnp.where(my == 0, local_v(b, h), vbuf[s0, b, h])
                v1 = jnp.where(my == 1, local_v(b, h), vbuf[s1, b, h])
                kband = jnp.concatenate([k0, k1[:BAND - SKV_SH]], axis=0)
                vband = jnp.concatenate([v0, v1[:BAND - SKV_SH]], axis=0)

                qband = qb[b * SQ + G:(b + 1) * SQ, h * DH:(h + 1) * DH]
                s = lax.dot_general(
                    qband, kband, (((1,), (1,)), ((), ())),
                    preferred_element_type=jnp.float32,
                ) * 0.125
                p = jnp.exp(jnp.where(band_mask, s, -1e9))
                pv = jnp.dot(p.astype(jnp.bfloat16), vband,
                             preferred_element_type=jnp.float32)
                den = jnp.sum(p, axis=1, keepdims=True)

                ctx_cols.append(jnp.concatenate(
                    [accg[b][h] / deng[b][h], pv / den], axis=0))
            ctx_rows.append(jnp.concatenate(ctx_cols, axis=1))
        ctx2d = jnp.concatenate(ctx_rows, axis=0)

        partial = lax.dot_general(
            ctx2d.astype(jnp.bfloat16), wo_ref[...].astype(jnp.bfloat16),
            (((1,), (0,)), ((), ())),
            preferred_element_type=jnp.float32,
        )
        pbuf[...] = partial.astype(jnp.bfloat16)

        p_rdmas = []
        for k in SEND_ORDER:
            dest = (my + k) % N_DEV
            r = pltpu.make_async_remote_copy(
                src_ref=pbuf,
                dst_ref=accbuf.at[k - 1],
                send_sem=send_p.at[k - 1],
                recv_sem=recv_p.at[k - 1],
                device_id=(dest,),
                device_id_type=pl.DeviceIdType.MESH,
            )
            r.start()
            p_rdmas.append(r)
        for r in p_rdmas:
            r.wait_recv()

        out_ref[...] = partial + (
            accbuf[0].astype(jnp.float32)
            + accbuf[1].astype(jnp.float32)
            + accbuf[2].astype(jnp.float32)
        )

        for k in SEND_ORDER:
            rk, rv = rdmas[k]
            rk.wait_send()
            rv.wait_send()
        for r in p_rdmas:
            r.wait_send()

    out2d = pl.pallas_call(
        body,
        out_shape=jax.ShapeDtypeStruct((B * SQ, D_MODEL), jnp.float32),
        in_specs=[pl.BlockSpec(memory_space=pltpu.VMEM)] * 5,
        out_specs=pl.BlockSpec(memory_space=pltpu.VMEM),
        scratch_shapes=[
            pltpu.VMEM((N_DEV - 1, B, H_LOC, SKV_SH, DH), jnp.bfloat16),
            pltpu.VMEM((N_DEV - 1, B, H_LOC, SKV_SH, DH), jnp.bfloat16),
            pltpu.VMEM((B * SQ, D_MODEL), jnp.bfloat16),
            pltpu.VMEM((N_DEV - 1, B * SQ, D_MODEL), jnp.bfloat16),
            pltpu.SemaphoreType.DMA((N_DEV - 1,)),
            pltpu.SemaphoreType.DMA((N_DEV - 1,)),
            pltpu.SemaphoreType.DMA((N_DEV - 1,)),
            pltpu.SemaphoreType.DMA((N_DEV - 1,)),
            pltpu.SemaphoreType.DMA((N_DEV - 1,)),
            pltpu.SemaphoreType.DMA((N_DEV - 1,)),
        ],
        compiler_params=pltpu.CompilerParams(collective_id=0),
    )(x2d, Wq, K_c, V_c, Wo)

    return out2d.reshape(B, SQ, D_MODEL)


# device time: 48089 ns/iter; 1.0871x vs baseline; 1.0871x over previous
import jax
import jax.numpy as jnp
from jax import lax
from jax.experimental import pallas as pl
from jax.experimental.pallas import tpu as pltpu

N_DEV = 4
B = 2
SQ = 256
SKV_SH = 256
SKV = N_DEV * SKV_SH
H_LOC = 4
DH = 64
D_MODEL = 512

G = 32
BAND = 384

SEND_ORDER = (2, 1, 3)
RECV_ORDER = (1, 3, 2)


def kernel(x, Wq, K_ext, V_ext, Wo):
    x2d = x.reshape(B * SQ, D_MODEL)
    K_c = K_ext.transpose(0, 2, 1, 3).astype(jnp.bfloat16)
    V_c = V_ext.transpose(0, 2, 1, 3).astype(jnp.bfloat16)

    def body(x_ref, wq_ref, k_ref, v_ref, wo_ref, out_ref,
             kbuf, vbuf, pall, rsbuf, agsrc, agbuf,
             send_k, recv_k, send_v, recv_v,
             send_rs, recv_rs, send_ag, recv_ag):
        my = lax.axis_index("i")

        bsem = pltpu.get_barrier_semaphore()
        for k in range(1, N_DEV):
            pl.semaphore_signal(
                bsem, inc=1,
                device_id=((my + k) % N_DEV,),
                device_id_type=pl.DeviceIdType.MESH,
            )
        pl.semaphore_wait(bsem, N_DEV - 1)

        rdmas = {}
        for k in SEND_ORDER:
            dest = (my + k) % N_DEV
            rk = pltpu.make_async_remote_copy(
                src_ref=k_ref.at[:, pl.ds(dest * H_LOC, H_LOC), :, :],
                dst_ref=kbuf.at[k - 1],
                send_sem=send_k.at[k - 1],
                recv_sem=recv_k.at[k - 1],
                device_id=(dest,),
                device_id_type=pl.DeviceIdType.MESH,
            )
            rk.start()
            rv = pltpu.make_async_remote_copy(
                src_ref=v_ref.at[:, pl.ds(dest * H_LOC, H_LOC), :, :],
                dst_ref=vbuf.at[k - 1],
                send_sem=send_v.at[k - 1],
                recv_sem=recv_v.at[k - 1],
                device_id=(dest,),
                device_id_type=pl.DeviceIdType.MESH,
            )
            rv.start()
            rdmas[k] = (rk, rv)

        qb = lax.dot_general(
            x_ref[...].astype(jnp.bfloat16), wq_ref[...].astype(jnp.bfloat16),
            (((1,), (0,)), ((), ())),
            preferred_element_type=jnp.float32,
        ).astype(jnp.bfloat16)

        def local_k(b, h):
            return k_ref[b, my * H_LOC + h]

        def local_v(b, h):
            return v_ref[b, my * H_LOC + h]

        accg = [[None] * H_LOC for _ in range(B)]
        deng = [[None] * H_LOC for _ in range(B)]

        def glob_rows(get_k, get_v, first):
            for b in range(B):
                for h in range(H_LOC):
                    qg = qb[b * SQ:b * SQ + G, h * DH:(h + 1) * DH]
                    s = lax.dot_general(
                        qg, get_k(b, h), (((1,), (1,)), ((), ())),
                        preferred_element_type=jnp.float32,
                    ) * 0.125
                    p = jnp.exp(s)
                    pv = jnp.dot(p.astype(jnp.bfloat16), get_v(b, h),
                                 preferred_element_type=jnp.float32)
                    rs = jnp.sum(p, axis=1, keepdims=True)
                    if first:
                        accg[b][h] = pv
                        deng[b][h] = rs
                    else:
                        accg[b][h] = accg[b][h] + pv
                        deng[b][h] = deng[b][h] + rs

        glob_rows(local_k, local_v, first=True)
        for k in RECV_ORDER:
            rk, rv = rdmas[k]
            rk.wait_recv()
            rv.wait_recv()
            glob_rows(lambda b, h, _k=k: kbuf[_k - 1, b, h],
                      lambda b, h, _k=k: vbuf[_k - 1, b, h],
                      first=False)

        s0 = jnp.maximum((my - 0) % N_DEV - 1, 0)
        s1 = jnp.maximum((my - 1) % N_DEV - 1, 0)

        qi3 = G + lax.broadcasted_iota(jnp.int32, (SQ - G, BAND), 0)
        ki3 = lax.broadcasted_iota(jnp.int32, (SQ - G, BAND), 1)
        band_mask = (jnp.abs(qi3 - ki3) <= 128) | (ki3 < G)

        ctx_rows = []
        for b in range(B):
            ctx_cols = []
            for h in range(H_LOC):
                k0 = jnp.where(my == 0, local_k(b, h), kbuf[s0, b, h])
                k1 = jnp.where(my == 1, local_k(b, h), kbuf[s1, b, h])
                v0 = jnp.where(my == 0, local_v(b, h), vbuf[s0, b, h])
                v1 = jnp.where(my == 1, local_v(b, h), vbuf[s1, b, h])
                kband = jnp.concatenate([k0, k1[:BAND - SKV_SH]], axis=0)
                vband = jnp.concatenate([v0, v1[:BAND - SKV_SH]], axis=0)

                qband = qb[b * SQ + G:(b + 1) * SQ, h * DH:(h + 1) * DH]
                s = lax.dot_general(
                    qband, kband, (((1,), (1,)), ((), ())),
                    preferred_element_type=jnp.float32,
                ) * 0.125
                p = jnp.exp(jnp.where(band_mask, s, -1e9))
                pv = jnp.dot(p.astype(jnp.bfloat16), vband,
                             preferred_element_type=jnp.float32)
                den = jnp.sum(p, axis=1, keepdims=True)

                ctx_cols.append(jnp.concatenate(
                    [accg[b][h] / deng[b][h], pv / den], axis=0))
            ctx_rows.append(jnp.concatenate(ctx_cols, axis=1))
        ctx2d = jnp.concatenate(ctx_rows, axis=0)

        RB = (B * SQ) // N_DEV
        partial = lax.dot_general(
            ctx2d.astype(jnp.bfloat16), wo_ref[...].astype(jnp.bfloat16),
            (((1,), (0,)), ((), ())),
            preferred_element_type=jnp.float32,
        )
        pall[...] = partial.astype(jnp.bfloat16).reshape(N_DEV, RB, D_MODEL)

        rs_rdmas = []
        for k in range(1, N_DEV):
            dest = (my + k) % N_DEV
            r = pltpu.make_async_remote_copy(
                src_ref=pall.at[dest],
                dst_ref=rsbuf.at[k - 1],
                send_sem=send_rs.at[k - 1],
                recv_sem=recv_rs.at[k - 1],
                device_id=(dest,),
                device_id_type=pl.DeviceIdType.MESH,
            )
            r.start()
            rs_rdmas.append(r)
        for r in rs_rdmas:
            r.wait_recv()

        red = pall[my].astype(jnp.float32) + (
            rsbuf[0].astype(jnp.float32)
            + rsbuf[1].astype(jnp.float32)
            + rsbuf[2].astype(jnp.float32)
        )
        out_ref[my] = red
        agsrc[...] = red.astype(jnp.bfloat16)

        ag_rdmas = []
        for k in range(1, N_DEV):
            dest = (my + k) % N_DEV
            r = pltpu.make_async_remote_copy(
                src_ref=agsrc,
                dst_ref=agbuf.at[k - 1],
                send_sem=send_ag.at[k - 1],
                recv_sem=recv_ag.at[k - 1],
                device_id=(dest,),
                device_id_type=pl.DeviceIdType.MESH,
            )
            r.start()
            ag_rdmas.append(r)
        for r in ag_rdmas:
            r.wait_recv()
        for k in range(1, N_DEV):
            origin = (my - k) % N_DEV
            out_ref[origin] = agbuf[k - 1].astype(jnp.float32)

        for k in SEND_ORDER:
            rk, rv = rdmas[k]
            rk.wait_send()
            rv.wait_send()
        for r in rs_rdmas:
            r.wait_send()
        for r in ag_rdmas:
            r.wait_send()

    out2d = pl.pallas_call(
        body,
        out_shape=jax.ShapeDtypeStruct((N_DEV, (B * SQ) // N_DEV, D_MODEL),
                                       jnp.float32),
        in_specs=[pl.BlockSpec(memory_space=pltpu.VMEM)] * 5,
        out_specs=pl.BlockSpec(memory_space=pltpu.VMEM),
        scratch_shapes=[
            pltpu.VMEM((N_DEV - 1, B, H_LOC, SKV_SH, DH), jnp.bfloat16),
            pltpu.VMEM((N_DEV - 1, B, H_LOC, SKV_SH, DH), jnp.bfloat16),
            pltpu.VMEM((N_DEV, (B * SQ) // N_DEV, D_MODEL),
                       jnp.bfloat16),
            pltpu.VMEM((N_DEV - 1, (B * SQ) // N_DEV, D_MODEL),
                       jnp.bfloat16),
            pltpu.VMEM(((B * SQ) // N_DEV, D_MODEL), jnp.bfloat16),
            pltpu.VMEM((N_DEV - 1, (B * SQ) // N_DEV, D_MODEL),
                       jnp.bfloat16),
            pltpu.SemaphoreType.DMA((N_DEV - 1,)),
            pltpu.SemaphoreType.DMA((N_DEV - 1,)),
            pltpu.SemaphoreType.DMA((N_DEV - 1,)),
            pltpu.SemaphoreType.DMA((N_DEV - 1,)),
            pltpu.SemaphoreType.DMA((N_DEV - 1,)),
            pltpu.SemaphoreType.DMA((N_DEV - 1,)),
            pltpu.SemaphoreType.DMA((N_DEV - 1,)),
            pltpu.SemaphoreType.DMA((N_DEV - 1,)),
        ],
        compiler_params=pltpu.CompilerParams(collective_id=0),
    )(x2d, Wq, K_c, V_c, Wo)

    return out2d.reshape(B, SQ, D_MODEL)


# device time: 47262 ns/iter; 1.1062x vs baseline; 1.0175x over previous
import jax
import jax.numpy as jnp
from jax import lax
from jax.experimental import pallas as pl
from jax.experimental.pallas import tpu as pltpu

N_DEV = 4
B = 2
SQ = 256
SKV_SH = 256
HQ = 16
H_LOC = 4
DH = 64
D_MODEL = 512

G = 32
BAND = 384
HB = BAND - SKV_SH
ND = DH + 16
RB = (B * SQ) // N_DEV


def kernel(x, Wq, K_ext, V_ext, Wo):
    x2d = x.reshape(B * SQ, D_MODEL)
    KV_c = jnp.stack(
        [K_ext.transpose(0, 2, 1, 3), V_ext.transpose(0, 2, 1, 3)], axis=0
    ).astype(jnp.bfloat16)

    def body(x_ref, wq_ref, kv_ref, wo_ref, out_ref,
             kvb0, kvb1, qgsrc, qgbuf, ndsrc, ndbuf,
             pall, rsbuf, agsrc, agbuf,
             send_band, recv_b0, recv_b1,
             send_qg, recv_qg, send_nd, recv_nd,
             send_rs, recv_rs, send_ag, recv_ag):
        my = lax.axis_index("i")

        bsem = pltpu.get_barrier_semaphore()
        for k in range(1, N_DEV):
            pl.semaphore_signal(
                bsem, inc=1,
                device_id=((my + k) % N_DEV,),
                device_id_type=pl.DeviceIdType.MESH,
            )
        pl.semaphore_wait(bsem, N_DEV - 1)

        wq16 = wq_ref[...].astype(jnp.bfloat16)
        xg = jnp.concatenate(
            [x_ref[b * SQ:b * SQ + G, :] for b in range(B)], axis=0
        ).astype(jnp.bfloat16)
        qg2d = lax.dot_general(
            xg, wq16, (((1,), (0,)), ((), ())),
            preferred_element_type=jnp.float32,
        ).astype(jnp.bfloat16)
        for b in range(B):
            for h in range(H_LOC):
                blk = qg2d[b * G:(b + 1) * G, h * DH:(h + 1) * DH]
                qgsrc[b, h] = blk
        qgbuf[:, pl.ds(my * H_LOC, H_LOC)] = qgsrc[...]

        qg_rdmas = []
        for k in range(1, N_DEV):
            dest = (my + k) % N_DEV
            r = pltpu.make_async_remote_copy(
                src_ref=qgsrc,
                dst_ref=qgbuf.at[:, pl.ds(my * H_LOC, H_LOC)],
                send_sem=send_qg.at[k - 1],
                recv_sem=recv_qg.at[k - 1],
                device_id=(dest,),
                device_id_type=pl.DeviceIdType.MESH,
            )
            r.start()
            qg_rdmas.append(r)

        @pl.when(my == 0)
        def _():
            for k in range(1, N_DEV):
                dest = (my + k) % N_DEV
                pltpu.make_async_remote_copy(
                    src_ref=kv_ref.at[:, :, pl.ds(dest * H_LOC, H_LOC), :, :],
                    dst_ref=kvb0,
                    send_sem=send_band.at[k - 1],
                    recv_sem=recv_b0,
                    device_id=(dest,),
                    device_id_type=pl.DeviceIdType.MESH,
                ).start()
            kvb0[...] = kv_ref[:, :, pl.ds(my * H_LOC, H_LOC), :, :]

        @pl.when(my == 1)
        def _():
            for k in range(1, N_DEV):
                dest = (my + k) % N_DEV
                pltpu.make_async_remote_copy(
                    src_ref=kv_ref.at[:, :, pl.ds(dest * H_LOC, H_LOC),
                                      pl.ds(0, HB), :],
                    dst_ref=kvb1,
                    send_sem=send_band.at[k - 1],
                    recv_sem=recv_b1,
                    device_id=(dest,),
                    device_id_type=pl.DeviceIdType.MESH,
                ).start()
            kvb1[...] = kv_ref[:, :, pl.ds(my * H_LOC, H_LOC), pl.ds(0, HB), :]

        qb = lax.dot_general(
            x_ref[...].astype(jnp.bfloat16), wq16,
            (((1,), (0,)), ((), ())),
            preferred_element_type=jnp.float32,
        ).astype(jnp.bfloat16)

        for r in qg_rdmas:
            r.wait_recv()
        for b in range(B):
            for hg in range(HQ):
                qga = qgbuf[b, hg]
                s = lax.dot_general(
                    qga, kv_ref[0, b, hg], (((1,), (1,)), ((), ())),
                    preferred_element_type=jnp.float32,
                ) * 0.125
                p = jnp.exp(s)
                num = jnp.dot(p.astype(jnp.bfloat16), kv_ref[1, b, hg],
                              preferred_element_type=jnp.float32)
                den = jnp.sum(p, axis=1, keepdims=True)
                ndsrc[b, hg] = jnp.concatenate(
                    [num, jnp.broadcast_to(den, (G, ND - DH))], axis=1
                ).astype(jnp.bfloat16)

        nd_rdmas = []
        for k in range(1, N_DEV):
            dest = (my + k) % N_DEV
            r = pltpu.make_async_remote_copy(
                src_ref=ndsrc.at[:, pl.ds(dest * H_LOC, H_LOC)],
                dst_ref=ndbuf.at[k - 1],
                send_sem=send_nd.at[k - 1],
                recv_sem=recv_nd.at[k - 1],
                device_id=(dest,),
                device_id_type=pl.DeviceIdType.MESH,
            )
            r.start()
            nd_rdmas.append(r)

        @pl.when(my != 0)
        def _():
            pltpu.make_async_remote_copy(
                src_ref=kv_ref.at[:, :, pl.ds(0, H_LOC), :, :],
                dst_ref=kvb0,
                send_sem=send_band.at[0],
                recv_sem=recv_b0,
                device_id=(0,),
                device_id_type=pl.DeviceIdType.MESH,
            ).wait_recv()

        @pl.when(my != 1)
        def _():
            pltpu.make_async_remote_copy(
                src_ref=kv_ref.at[:, :, pl.ds(0, H_LOC), pl.ds(0, HB), :],
                dst_ref=kvb1,
                send_sem=send_band.at[0],
                recv_sem=recv_b1,
                device_id=(1,),
                device_id_type=pl.DeviceIdType.MESH,
            ).wait_recv()

        qi3 = G + lax.broadcasted_iota(jnp.int32, (SQ - G, BAND), 0)
        ki3 = lax.broadcasted_iota(jnp.int32, (SQ - G, BAND), 1)
        band_mask = (jnp.abs(qi3 - ki3) <= 128) | (ki3 < G)

        band_ctx = [[None] * H_LOC for _ in range(B)]
        for b in range(B):
            for h in range(H_LOC):
                kband = jnp.concatenate(
                    [kvb0[0, b, h], kvb1[0, b, h]], axis=0)
                vband = jnp.concatenate(
                    [kvb0[1, b, h], kvb1[1, b, h]], axis=0)
                qband = qb[b * SQ + G:(b + 1) * SQ, h * DH:(h + 1) * DH]
                s = lax.dot_general(
                    qband, kband, (((1,), (1,)), ((), ())),
                    preferred_element_type=jnp.float32,
                ) * 0.125
                p = jnp.exp(jnp.where(band_mask, s, -1e9))
                pv = jnp.dot(p.astype(jnp.bfloat16), vband,
                             preferred_element_type=jnp.float32)
                den = jnp.sum(p, axis=1, keepdims=True)
                band_ctx[b][h] = pv / den

        for r in nd_rdmas:
            r.wait_recv()
        ctx_rows = []
        for b in range(B):
            ctx_cols = []
            for h in range(H_LOC):
                pack = ndsrc[b, my * H_LOC + h].astype(jnp.float32)
                for k in range(1, N_DEV):
                    pack = pack + ndbuf[k - 1, b, h].astype(jnp.float32)
                glob_ctx = pack[:, :DH] / pack[:, DH:DH + 1]
                ctx_cols.append(jnp.concatenate(
                    [glob_ctx, band_ctx[b][h]], axis=0))
            ctx_rows.append(jnp.concatenate(ctx_cols, axis=1))
        ctx2d = jnp.concatenate(ctx_rows, axis=0)

        partial = lax.dot_general(
            ctx2d.astype(jnp.bfloat16), wo_ref[...].astype(jnp.bfloat16),
            (((1,), (0,)), ((), ())),
            preferred_element_type=jnp.float32,
        )
        pall[...] = partial.astype(jnp.bfloat16).reshape(N_DEV, RB, D_MODEL)

        rs_rdmas = []
        for k in range(1, N_DEV):
            dest = (my + k) % N_DEV
            r = pltpu.make_async_remote_copy(
                src_ref=pall.at[dest],
                dst_ref=rsbuf.at[k - 1],
                send_sem=send_rs.at[k - 1],
                recv_sem=recv_rs.at[k - 1],
                device_id=(dest,),
                device_id_type=pl.DeviceIdType.MESH,
            )
            r.start()
            rs_rdmas.append(r)
        for r in rs_rdmas:
            r.wait_recv()

        red = pall[my].astype(jnp.float32) + (
            rsbuf[0].astype(jnp.float32)
            + rsbuf[1].astype(jnp.float32)
            + rsbuf[2].astype(jnp.float32)
        )
        out_ref[my] = red
        agsrc[...] = red.astype(jnp.bfloat16)

        ag_rdmas = []
        for k in range(1, N_DEV):
            dest = (my + k) % N_DEV
            r = pltpu.make_async_remote_copy(
                src_ref=agsrc,
                dst_ref=agbuf.at[k - 1],
                send_sem=send_ag.at[k - 1],
                recv_sem=recv_ag.at[k - 1],
                device_id=(dest,),
                device_id_type=pl.DeviceIdType.MESH,
            )
            r.start()
            ag_rdmas.append(r)
        for r in ag_rdmas:
            r.wait_recv()
        for k in range(1, N_DEV):
            origin = (my - k) % N_DEV
            out_ref[origin] = agbuf[k - 1].astype(jnp.float32)

        for r in qg_rdmas + nd_rdmas + rs_rdmas + ag_rdmas:
            r.wait_send()

        @pl.when(my == 0)
        def _():
            for k in range(1, N_DEV):
                pltpu.make_async_remote_copy(
                    src_ref=kv_ref.at[:, :, pl.ds(0, H_LOC), :, :],
                    dst_ref=kvb0,
                    send_sem=send_band.at[k - 1],
                    recv_sem=recv_b0,
                    device_id=(0,),
                    device_id_type=pl.DeviceIdType.MESH,
                ).wait_send()

        @pl.when(my == 1)
        def _():
            for k in range(1, N_DEV):
                pltpu.make_async_remote_copy(
                    src_ref=kv_ref.at[:, :, pl.ds(0, H_LOC), pl.ds(0, HB), :],
                    dst_ref=kvb1,
                    send_sem=send_band.at[k - 1],
                    recv_sem=recv_b1,
                    device_id=(1,),
                    device_id_type=pl.DeviceIdType.MESH,
                ).wait_send()

    out3d = pl.pallas_call(
        body,
        out_shape=jax.ShapeDtypeStruct((N_DEV, RB, D_MODEL), jnp.float32),
        in_specs=[pl.BlockSpec(memory_space=pltpu.VMEM)] * 4,
        out_specs=pl.BlockSpec(memory_space=pltpu.VMEM),
        scratch_shapes=[
            pltpu.VMEM((2, B, H_LOC, SKV_SH, DH), jnp.bfloat16),
            pltpu.VMEM((2, B, H_LOC, HB, DH), jnp.bfloat16),
            pltpu.VMEM((B, H_LOC, G, DH), jnp.bfloat16),
            pltpu.VMEM((B, HQ, G, DH), jnp.bfloat16),
            pltpu.VMEM((B, HQ, G, ND), jnp.bfloat16),
            pltpu.VMEM((N_DEV - 1, B, H_LOC, G, ND), jnp.bfloat16),
            pltpu.VMEM((N_DEV, RB, D_MODEL), jnp.bfloat16),
            pltpu.VMEM((N_DEV - 1, RB, D_MODEL), jnp.bfloat16),
            pltpu.VMEM((RB, D_MODEL), jnp.bfloat16),
            pltpu.VMEM((N_DEV - 1, RB, D_MODEL), jnp.bfloat16),
            pltpu.SemaphoreType.DMA((N_DEV - 1,)),
            pltpu.SemaphoreType.DMA,
            pltpu.SemaphoreType.DMA,
            pltpu.SemaphoreType.DMA((N_DEV - 1,)),
            pltpu.SemaphoreType.DMA((N_DEV - 1,)),
            pltpu.SemaphoreType.DMA((N_DEV - 1,)),
            pltpu.SemaphoreType.DMA((N_DEV - 1,)),
            pltpu.SemaphoreType.DMA((N_DEV - 1,)),
            pltpu.SemaphoreType.DMA((N_DEV - 1,)),
            pltpu.SemaphoreType.DMA((N_DEV - 1,)),
            pltpu.SemaphoreType.DMA((N_DEV - 1,)),
        ],
        compiler_params=pltpu.CompilerParams(collective_id=0),
    )(x2d, Wq, KV_c, Wo)

    return out3d.reshape(B, SQ, D_MODEL)
